# baseline (speedup 1.0000x reference)
import functools

import jax
import jax.numpy as jnp
from jax import lax
from jax.experimental import pallas as pl
import jax.experimental.pallas.tpu as pltpu
from jax.experimental.pallas import tpu_sc as plsc

B, H, W = 32, 512, 512
NC, NS, L = 2, 16, 16
NW = NC * NS
RS = H // NW
NG = W // L
NSLOT = 4
BPK = 4


def _sc_kernel(in_hbm, seg_hbm, rowpart_hbm, colpart_hbm,
               bufs, red, rowbuf, colbuf, in_sems, out_sems):
    c = lax.axis_index("c")
    s = lax.axis_index("s")
    wid = s * NC + c
    base = wid * RS

    def in_copy(b, slot):
        return pltpu.async_copy(
            in_hbm.at[b, pl.ds(base, RS)], bufs.at[slot], in_sems.at[slot])

    def out_copy(b, slot):
        return pltpu.async_copy(
            bufs.at[slot], seg_hbm.at[b, pl.ds(base, RS)], out_sems.at[slot])

    zeros = jnp.zeros((L,), jnp.float32)
    for r in range(RS):
        for g in range(NG):
            red[r, pl.ds(g * L, L)] = zeros

    in_copy(0, 0).start()
    in_copy(1, 1).start()

    def body(k, carry):
        for j in range(BPK):
            t = BPK * k + j
            if j in (0, 1):
                @pl.when(k == 0)
                def _pre_a0():
                    in_copy(t + 2, (j + 2) % NSLOT).start()

                @pl.when(k >= 1)
                def _pre_a():
                    out_copy(t - 2, (j + 2) % NSLOT).wait()
                    in_copy(t + 2, (j + 2) % NSLOT).start()
            else:
                @pl.when(k < (B // BPK) - 1)
                def _pre_b():
                    out_copy(t - 2, (j + 2) % NSLOT).wait()
                    in_copy(t + 2, (j + 2) % NSLOT).start()

                @pl.when(k == (B // BPK) - 1)
                def _pre_c():
                    out_copy(t - 2, (j + 2) % NSLOT).wait()

            in_copy(t, j).wait()
            for r in range(RS):
                for g in range(NG):
                    plsc.addupdate(red.at[r, pl.ds(g * L, L)],
                                   bufs[j, r, pl.ds(g * L, L)])
            out_copy(t, j).start()
        return carry

    lax.fori_loop(0, B // BPK, body, 0)
    out_copy(B - 2, (B - 2) % NSLOT).wait()
    out_copy(B - 1, (B - 1) % NSLOT).wait()

    rowregs = [None] * RS
    for g in range(NG):
        creg = red[0, pl.ds(g * L, L)]
        rowregs[0] = creg if g == 0 else rowregs[0] + creg
        for r in range(1, RS):
            v = red[r, pl.ds(g * L, L)]
            creg = creg + v
            rowregs[r] = v if g == 0 else rowregs[r] + v
        colbuf[0, pl.ds(g * L, L)] = creg
    for r in range(RS):
        rowbuf[r, :] = rowregs[r]
    pltpu.sync_copy(rowbuf, rowpart_hbm.at[pl.ds(base, RS)])
    pltpu.sync_copy(colbuf, colpart_hbm.at[pl.ds(wid, 1)])


_sc_call = functools.partial(
    pl.kernel,
    out_type=[
        jax.ShapeDtypeStruct((B, H, W), jnp.float32),
        jax.ShapeDtypeStruct((H, L), jnp.float32),
        jax.ShapeDtypeStruct((NW, W), jnp.float32),
    ],
    mesh=plsc.VectorSubcoreMesh(core_axis_name="c", subcore_axis_name="s"),
    scratch_types=[
        pltpu.VMEM((NSLOT, RS, W), jnp.float32),
        pltpu.VMEM((RS, W), jnp.float32),
        pltpu.VMEM((RS, L), jnp.float32),
        pltpu.VMEM((1, W), jnp.float32),
        pltpu.SemaphoreType.DMA((NSLOT,)),
        pltpu.SemaphoreType.DMA((NSLOT,)),
    ],
)(_sc_kernel)


def _det_kernel(rowpart_ref, colpart_ref, det_ref):
    rowv = jnp.sum(rowpart_ref[...], axis=1, keepdims=True)
    colv = jnp.sum(colpart_ref[...], axis=0, keepdims=True)
    hidx = jax.lax.broadcasted_iota(jnp.int32, (H, 1), 0)
    widx = jax.lax.broadcasted_iota(jnp.int32, (1, W), 1)
    has = jnp.max(rowv) > 0.0
    ymin = jnp.min(jnp.where(rowv > 0.0, hidx, H))
    ymax = jnp.max(jnp.where(rowv > 0.0, hidx, -1))
    xmin = jnp.min(jnp.where(colv > 0.0, widx, W))
    xmax = jnp.max(jnp.where(colv > 0.0, widx, -1))
    ymin = jnp.where(has, ymin, 0)
    ymax = jnp.where(has, ymax, 0)
    xmin = jnp.where(has, xmin, 0)
    xmax = jnp.where(has, xmax, 0)
    height = ymax - ymin
    width = xmax - xmin
    cy = ymin + height // 2
    cx = xmin + width // 2
    conf = jnp.clip(100 * height * width, 0, 100)
    lane = jax.lax.broadcasted_iota(jnp.int32, (8, 128), 1)
    det = jnp.where(lane == 0, cx,
          jnp.where(lane == 1, cy,
          jnp.where(lane == 2, width,
          jnp.where(lane == 3, height,
          jnp.where(lane == 4, 5,
          jnp.where(lane == 5, conf, 0))))))
    det_ref[...] = det


def kernel(inputs):
    seg3, rowpart, colpart = _sc_call(inputs)
    det_pad = pl.pallas_call(
        _det_kernel,
        out_shape=jax.ShapeDtypeStruct((8, 128), jnp.int32),
    )(rowpart, colpart)
    det = jnp.broadcast_to(det_pad[0, :6][None, None, :], (B, 1, 6))
    return det, seg3[:, None]


# speedup vs baseline: 4.6371x; 4.6371x over previous
import jax
import jax.numpy as jnp
from jax.experimental import pallas as pl
import jax.experimental.pallas.tpu as pltpu

B, H, W = 32, 512, 512
CHUNK = 8
SLOTS = 4
LOOKAHEAD = 2
NSTEPS = B // CHUNK


def _bbox_kernel(in_hbm, seg_hbm, det_ref, bufs, acc, in_sems, out_sems):
    def in_copy(i, slot):
        return pltpu.make_async_copy(
            in_hbm.at[pl.ds(i * CHUNK, CHUNK)],
            bufs.at[slot],
            in_sems.at[slot],
        )

    def out_copy(i, slot):
        return pltpu.make_async_copy(
            bufs.at[slot],
            seg_hbm.at[pl.ds(i * CHUNK, CHUNK), 0],
            out_sems.at[slot],
        )

    acc[...] = jnp.zeros((H, W), jnp.float32)
    for p in range(LOOKAHEAD):
        in_copy(p, p).start()

    def step(i, _):
        s = jax.lax.rem(i, SLOTS)
        in_copy(i, s).wait()
        out_copy(i, s).start()
        x = bufs[s]
        acc[...] = jnp.maximum(acc[...], jnp.max(jnp.abs(x), axis=0))

        @pl.when(i + LOOKAHEAD < NSTEPS)
        def _prefetch():
            nxt = i + LOOKAHEAD
            s2 = jax.lax.rem(nxt, SLOTS)

            @pl.when(nxt >= SLOTS)
            def _reclaim():
                out_copy(nxt - SLOTS, s2).wait()

            in_copy(nxt, s2).start()

        return 0

    jax.lax.fori_loop(0, NSTEPS, step, 0)

    def drain(i, _):
        c = NSTEPS - SLOTS + i
        out_copy(c, jax.lax.rem(c, SLOTS)).wait()
        return 0

    jax.lax.fori_loop(0, SLOTS, drain, 0)

    m = acc[...]
    rm = jnp.max(m, axis=1, keepdims=True)
    cm = jnp.max(m, axis=0, keepdims=True)
    hidx = jax.lax.broadcasted_iota(jnp.int32, (H, 1), 0)
    widx = jax.lax.broadcasted_iota(jnp.int32, (1, W), 1)
    has = jnp.max(rm) > 0.0
    ymin = jnp.min(jnp.where(rm > 0.0, hidx, H))
    ymax = jnp.max(jnp.where(rm > 0.0, hidx, -1))
    xmin = jnp.min(jnp.where(cm > 0.0, widx, W))
    xmax = jnp.max(jnp.where(cm > 0.0, widx, -1))
    ymin = jnp.where(has, ymin, 0)
    ymax = jnp.where(has, ymax, 0)
    xmin = jnp.where(has, xmin, 0)
    xmax = jnp.where(has, xmax, 0)
    height = ymax - ymin
    width = xmax - xmin
    cy = ymin + height // 2
    cx = xmin + width // 2
    conf = jnp.clip(100 * height * width, 0, 100)
    lane = jax.lax.broadcasted_iota(jnp.int32, (8, 128), 1)
    det = jnp.where(lane == 0, cx,
          jnp.where(lane == 1, cy,
          jnp.where(lane == 2, width,
          jnp.where(lane == 3, height,
          jnp.where(lane == 4, 5,
          jnp.where(lane == 5, conf, 0))))))
    det_ref[...] = det


def kernel(inputs):
    seg, det_pad = pl.pallas_call(
        _bbox_kernel,
        in_specs=[pl.BlockSpec(memory_space=pltpu.MemorySpace.HBM)],
        out_specs=[
            pl.BlockSpec(memory_space=pltpu.MemorySpace.HBM),
            pl.BlockSpec(memory_space=pltpu.MemorySpace.VMEM),
        ],
        out_shape=[
            jax.ShapeDtypeStruct((B, 1, H, W), jnp.float32),
            jax.ShapeDtypeStruct((8, 128), jnp.int32),
        ],
        scratch_shapes=[
            pltpu.VMEM((SLOTS, CHUNK, H, W), jnp.float32),
            pltpu.VMEM((H, W), jnp.float32),
            pltpu.SemaphoreType.DMA((SLOTS,)),
            pltpu.SemaphoreType.DMA((SLOTS,)),
        ],
    )(inputs)
    det = jnp.broadcast_to(det_pad[0, :6][None, None, :], (B, 1, 6))
    return det, seg


# speedup vs baseline: 4.6910x; 1.0116x over previous
import jax
import jax.numpy as jnp
from jax.experimental import pallas as pl
import jax.experimental.pallas.tpu as pltpu

B, H, W = 32, 512, 512
CHUNK = 8
SLOTS = 4
LOOKAHEAD = 2
NSTEPS = B // CHUNK


def _bbox_kernel(in_hbm, seg_hbm, det_ref, bufs, acc, in_sems, out_sems):
    def in_copy(i, slot):
        return pltpu.make_async_copy(
            in_hbm.at[pl.ds(i * CHUNK, CHUNK)],
            bufs.at[slot],
            in_sems.at[slot],
        )

    def out_copy(i, slot):
        return pltpu.make_async_copy(
            bufs.at[slot],
            seg_hbm.at[pl.ds(i * CHUNK, CHUNK), 0],
            out_sems.at[slot],
        )

    for p in range(LOOKAHEAD):
        in_copy(p, p).start()
    acc[...] = jnp.zeros((H, W), jnp.float32)

    def step(i, _):
        s = jax.lax.rem(i, SLOTS)
        in_copy(i, s).wait()
        out_copy(i, s).start()
        x = bufs[s]
        acc[...] = jnp.maximum(acc[...], jnp.max(jnp.abs(x), axis=0))

        @pl.when(i + LOOKAHEAD < NSTEPS)
        def _prefetch():
            nxt = i + LOOKAHEAD
            s2 = jax.lax.rem(nxt, SLOTS)

            @pl.when(nxt >= SLOTS)
            def _reclaim():
                out_copy(nxt - SLOTS, s2).wait()

            in_copy(nxt, s2).start()

        return 0

    jax.lax.fori_loop(0, NSTEPS, step, 0)

    m = acc[...]
    rm = jnp.max(m, axis=1, keepdims=True)
    cm = jnp.max(m, axis=0, keepdims=True)
    hidx = jax.lax.broadcasted_iota(jnp.int32, (H, 1), 0)
    widx = jax.lax.broadcasted_iota(jnp.int32, (1, W), 1)
    has = jnp.max(rm) > 0.0
    ymin = jnp.min(jnp.where(rm > 0.0, hidx, H))
    ymax = jnp.max(jnp.where(rm > 0.0, hidx, -1))
    xmin = jnp.min(jnp.where(cm > 0.0, widx, W))
    xmax = jnp.max(jnp.where(cm > 0.0, widx, -1))
    ymin = jnp.where(has, ymin, 0)
    ymax = jnp.where(has, ymax, 0)
    xmin = jnp.where(has, xmin, 0)
    xmax = jnp.where(has, xmax, 0)
    height = ymax - ymin
    width = xmax - xmin
    cy = ymin + height // 2
    cx = xmin + width // 2
    conf = jnp.clip(100 * height * width, 0, 100)
    lane = jax.lax.broadcasted_iota(jnp.int32, (8, 128), 1)
    det = jnp.where(lane == 0, cx,
          jnp.where(lane == 1, cy,
          jnp.where(lane == 2, width,
          jnp.where(lane == 3, height,
          jnp.where(lane == 4, 5,
          jnp.where(lane == 5, conf, 0))))))
    det_ref[...] = det

    def drain(i, _):
        c = NSTEPS - SLOTS + i
        out_copy(c, jax.lax.rem(c, SLOTS)).wait()
        return 0

    jax.lax.fori_loop(0, SLOTS, drain, 0)


def kernel(inputs):
    seg, det_pad = pl.pallas_call(
        _bbox_kernel,
        in_specs=[pl.BlockSpec(memory_space=pltpu.MemorySpace.HBM)],
        out_specs=[
            pl.BlockSpec(memory_space=pltpu.MemorySpace.HBM),
            pl.BlockSpec(memory_space=pltpu.MemorySpace.VMEM),
        ],
        out_shape=[
            jax.ShapeDtypeStruct((B, 1, H, W), jnp.float32),
            jax.ShapeDtypeStruct((8, 128), jnp.int32),
        ],
        scratch_shapes=[
            pltpu.VMEM((SLOTS, CHUNK, H, W), jnp.float32),
            pltpu.VMEM((H, W), jnp.float32),
            pltpu.SemaphoreType.DMA((SLOTS,)),
            pltpu.SemaphoreType.DMA((SLOTS,)),
        ],
    )(inputs)
    det = jnp.broadcast_to(det_pad[0, :6][None, None, :], (B, 1, 6))
    return det, seg
